# baseline (speedup 1.0000x reference)
import functools
import math

import jax
import jax.numpy as jnp
from jax import lax
from jax.experimental import pallas as pl
from jax.experimental.pallas import tpu as pltpu
from jax.experimental.pallas import tpu_sc as plsc

VOCAB = 1000000
D = 64
SCALE = math.sqrt(D)

NC = 2
NS = 16
NW = NC * NS

B = 16384 * 20
B_PER_W = B // NW
CHUNK = 128
NCH = B_PER_W // CHUNK
LPR = D // 16


@functools.partial(
    pl.kernel,
    mesh=plsc.VectorSubcoreMesh(core_axis_name="c", subcore_axis_name="s"),
    out_type=jax.ShapeDtypeStruct((B, D), jnp.float32),
    scratch_types=[
        pltpu.VMEM((NCH, CHUNK), jnp.int32),
        pltpu.VMEM((CHUNK, D), jnp.float32),
        pltpu.SemaphoreType.DMA,
    ],
    compiler_params=pltpu.CompilerParams(use_tc_tiling_on_sc=False),
)
def _embed_kernel(idx_hbm, table_hbm, out_hbm, idx_v, rows_v, sem):
    cid = lax.axis_index("c")
    sid = lax.axis_index("s")
    wid = sid * NC + cid
    pltpu.sync_copy(idx_hbm.at[wid], idx_v)
    base = wid * B_PER_W

    def chunk_body(ci, carry):
        pltpu.async_copy(table_hbm.at[idx_v.at[ci]], rows_v, sem).wait()

        def scale_row(r, c2):
            for j in range(LPR):
                sl = pl.ds(j * 16, 16)
                rows_v[r, sl] = rows_v[r, sl] * SCALE
            return c2

        lax.fori_loop(0, CHUNK, scale_row, 0)
        pltpu.sync_copy(rows_v, out_hbm.at[pl.ds(base + ci * CHUNK, CHUNK)])
        return carry

    lax.fori_loop(0, NCH, chunk_body, 0)


def kernel(x, table):
    idx = x.astype(jnp.int32).reshape(NW, NCH, CHUNK)
    out = _embed_kernel(idx, table)
    return out.reshape(x.shape[0], x.shape[1], D)


# speedup vs baseline: 1.1145x; 1.1145x over previous
import functools
import math

import jax
import jax.numpy as jnp
from jax import lax
from jax.experimental import pallas as pl
from jax.experimental.pallas import tpu as pltpu
from jax.experimental.pallas import tpu_sc as plsc

VOCAB = 1000000
D = 64
SCALE = math.sqrt(D)

NC = 2
NS = 16
NW = NC * NS

B = 16384 * 20
B_PER_W = B // NW
CHUNK = 128
NCH = B_PER_W // CHUNK
LPR = D // 16
NBUF = 4


@functools.partial(
    pl.kernel,
    mesh=plsc.VectorSubcoreMesh(core_axis_name="c", subcore_axis_name="s"),
    out_type=jax.ShapeDtypeStruct((B, D), jnp.float32),
    scratch_types=[
        pltpu.VMEM((NCH, CHUNK), jnp.int32),
        pltpu.VMEM((NBUF, CHUNK, D), jnp.float32),
        pltpu.VMEM((NBUF, CHUNK, D), jnp.float32),
        pltpu.SemaphoreType.DMA((NBUF,)),
        pltpu.SemaphoreType.DMA((NBUF,)),
    ],
    compiler_params=pltpu.CompilerParams(use_tc_tiling_on_sc=False),
)
def _embed_kernel(idx_hbm, table_hbm, out_hbm, idx_v, gbuf, sbuf, gsem, ssem):
    cid = lax.axis_index("c")
    sid = lax.axis_index("s")
    wid = sid * NC + cid
    pltpu.sync_copy(idx_hbm.at[wid], idx_v)
    base = wid * B_PER_W

    def start_gather(c, b):
        pltpu.async_copy(table_hbm.at[idx_v.at[c]], gbuf.at[b], gsem.at[b])

    def wait_gather(c, b):
        pltpu.make_async_copy(table_hbm.at[idx_v.at[c]], gbuf.at[b],
                              gsem.at[b]).wait()

    def out_slot(c):
        return out_hbm.at[pl.ds(base + c * CHUNK, CHUNK)]

    def start_scatter(c, b):
        pltpu.async_copy(sbuf.at[b], out_slot(c), ssem.at[b])

    def wait_scatter(c, b):
        pltpu.make_async_copy(sbuf.at[b], out_slot(c), ssem.at[b]).wait()

    def scale_chunk(b):
        def scale_row(r, carry):
            for j in range(LPR):
                sl = pl.ds(j * 16, 16)
                sbuf[b, r, sl] = gbuf[b, r, sl] * SCALE
            return carry

        lax.fori_loop(0, CHUNK, scale_row, 0)

    for b in range(NBUF):
        start_gather(b, b)

    for b in range(NBUF):
        wait_gather(b, b)
        scale_chunk(b)
        start_gather(b + NBUF, b)
        start_scatter(b, b)

    @pl.loop(NBUF, NCH - NBUF, step=NBUF)
    def _steady(ci):
        for b in range(NBUF):
            c = ci + b
            wait_gather(c, b)
            wait_scatter(c - NBUF, b)
            scale_chunk(b)
            start_gather(c + NBUF, b)
            start_scatter(c, b)

    for b in range(NBUF):
        c = NCH - NBUF + b
        wait_gather(c, b)
        wait_scatter(c - NBUF, b)
        scale_chunk(b)
        start_scatter(c, b)

    for b in range(NBUF):
        wait_scatter(NCH - NBUF + b, b)


def kernel(x, table):
    idx = x.astype(jnp.int32).reshape(NW, NCH, CHUNK)
    out = _embed_kernel(idx, table)
    return out.reshape(x.shape[0], x.shape[1], D)
